# Initial kernel scaffold; baseline (speedup 1.0000x reference)
import jax
import jax.numpy as jnp
from jax.experimental import pallas as pl


def kernel(outputs, classes, emb, ids):
    raise NotImplementedError("write your pallas kernel here")


# speedup vs baseline: 622.9082x; 622.9082x over previous
import jax
import jax.numpy as jnp
from jax.experimental import pallas as pl

ALPHA = 0.5
BETA = 0.5
GAMMA = 1.0
M_MARGIN = 2.0
S_SCALE = 64.0
ANG_MARGIN = 0.75
EPS = 1e-6
B = 512
D = 256


def _loss_body(out_ref, cls_ref, emb_ref, ids_ref, res_ref):
    x = out_ref[:]
    c = cls_ref[:]
    emb = emb_ref[:]
    ids = ids_ref[:]

    sin_t = jnp.sqrt(jnp.maximum(1.0 - x * x, 0.0))
    cos_m = jnp.float32(jnp.cos(ANG_MARGIN))
    sin_m = jnp.float32(jnp.sin(ANG_MARGIN))
    logits = (x * cos_m - (2.0 * c - 1.0) * sin_t * sin_m) * S_SCALE
    bce_terms = (jnp.maximum(logits, 0.0) - logits * c
                 + jnp.log1p(jnp.exp(-jnp.abs(logits))))
    bce = jnp.sum(bce_terms) * (1.0 / B)

    gram = jax.lax.dot_general(
        emb, emb, (((1,), (1,)), ((), ())),
        preferred_element_type=jnp.float32)
    nrm = jnp.sum(emb * emb, axis=1, keepdims=True)
    rsum = jnp.sum(emb, axis=1, keepdims=True)
    d2 = (nrm + nrm.T - 2.0 * gram
          + (2.0 * EPS) * (rsum - rsum.T) + (D * EPS * EPS))
    dist = jnp.sqrt(jnp.maximum(d2, 0.0))

    row = jax.lax.broadcasted_iota(jnp.int32, (B, B), 0)
    col = jax.lax.broadcasted_iota(jnp.int32, (B, B), 1)
    upper = row < col
    id_eq = ids == ids.T
    cls_eq = c == c.T
    c1 = jnp.logical_and(upper, jnp.logical_and(~id_eq, cls_eq))
    c2 = jnp.logical_and(upper, jnp.logical_and(id_eq, ~cls_eq))
    c1f = c1.astype(jnp.float32)
    c2f = c2.astype(jnp.float32)

    n1 = jnp.sum(c1f)
    n2 = jnp.sum(c2f)
    t1 = jnp.sum(dist * c1f) / jnp.maximum(n1, 1.0)
    t2 = jnp.sum(jnp.maximum(0.0, M_MARGIN - dist) * c2f) / jnp.maximum(n2, 1.0)

    loss = (GAMMA * bce
            + jnp.where(n1 > 0, ALPHA * t1, 0.0)
            + jnp.where(n2 > 0, BETA * t2, 0.0))
    res_ref[:] = jnp.full((1, 1), loss, dtype=jnp.float32)


def kernel(outputs, classes, emb, ids):
    ids2d = ids.reshape(B, 1).astype(jnp.int32)
    res = pl.pallas_call(
        _loss_body,
        out_shape=jax.ShapeDtypeStruct((1, 1), jnp.float32),
    )(outputs, classes, emb, ids2d)
    return res.reshape(())


# speedup vs baseline: 681.4149x; 1.0939x over previous
import jax
import jax.numpy as jnp
from jax.experimental import pallas as pl

ALPHA = 0.5
BETA = 0.5
GAMMA = 1.0
M_MARGIN = 2.0
S_SCALE = 64.0
ANG_MARGIN = 0.75
EPS = 1e-6
B = 512
D = 256


def _loss_body(out_ref, cls_ref, emb_ref, ids_ref, res_ref):
    emb = emb_ref[:]
    c_col = cls_ref[:]
    ids_col = ids_ref[:]
    x_row = out_ref[:].T
    c_row = c_col.T
    ids_row = ids_col.T

    sin_t = jnp.sqrt(jnp.maximum(1.0 - x_row * x_row, 0.0))
    cos_m = jnp.float32(jnp.cos(ANG_MARGIN))
    sin_m = jnp.float32(jnp.sin(ANG_MARGIN))
    logits = (x_row * cos_m - (2.0 * c_row - 1.0) * sin_t * sin_m) * S_SCALE
    bce_terms = (jnp.maximum(logits, 0.0) - logits * c_row
                 + jnp.log1p(jnp.exp(-jnp.abs(logits))))
    bce = jnp.sum(bce_terms) * (1.0 / B)

    ones_d = jnp.ones((D, 1), dtype=jnp.float32)
    nrm = jax.lax.dot_general(emb * emb, ones_d, (((1,), (0,)), ((), ())),
                              preferred_element_type=jnp.float32)
    rsum = jax.lax.dot_general(emb, ones_d, (((1,), (0,)), ((), ())),
                               preferred_element_type=jnp.float32)
    aux1 = nrm + (2.0 * EPS) * rsum + (D * EPS * EPS)
    aux2 = nrm - (2.0 * EPS) * rsum
    ones_col = jnp.ones((B, 1), dtype=jnp.float32)
    u = jnp.concatenate([emb * -2.0, aux1, ones_col], axis=1)
    v = jnp.concatenate([emb, ones_col, aux2], axis=1)
    d2 = jax.lax.dot_general(u, v, (((1,), (1,)), ((), ())),
                             preferred_element_type=jnp.float32)
    dist = jnp.sqrt(jnp.maximum(d2, 0.0))

    id_eq = ids_col == ids_row
    cls_eq = c_col == c_row
    c1 = jnp.logical_and(~id_eq, cls_eq)
    c2 = jnp.logical_and(id_eq, ~cls_eq)

    s1 = jnp.sum(jnp.where(c1, dist, 0.0)) * 0.5
    n1 = jnp.sum(jnp.where(c1, 1.0, 0.0)) * 0.5
    s2 = jnp.sum(jnp.where(c2, jnp.maximum(0.0, M_MARGIN - dist), 0.0)) * 0.5
    n2 = jnp.sum(jnp.where(c2, 1.0, 0.0)) * 0.5

    t1 = s1 / jnp.maximum(n1, 1.0)
    t2 = s2 / jnp.maximum(n2, 1.0)
    loss = (GAMMA * bce
            + jnp.where(n1 > 0, ALPHA * t1, 0.0)
            + jnp.where(n2 > 0, BETA * t2, 0.0))
    res_ref[:] = jnp.full((1, 1), loss, dtype=jnp.float32)


def kernel(outputs, classes, emb, ids):
    ids2d = ids.reshape(B, 1).astype(jnp.int32)
    res = pl.pallas_call(
        _loss_body,
        out_shape=jax.ShapeDtypeStruct((1, 1), jnp.float32),
    )(outputs, classes, emb, ids2d)
    return res.reshape(())


# speedup vs baseline: 768.9891x; 1.1285x over previous
import jax
import jax.numpy as jnp
from jax.experimental import pallas as pl

ALPHA = 0.5
BETA = 0.5
GAMMA = 1.0
M_MARGIN = 2.0
S_SCALE = 64.0
ANG_MARGIN = 0.75
EPS = 1e-6
B = 512
D = 256


def _loss_body(out_ref, cls_ref, emb_ref, ids_ref, res_ref):
    emb = emb_ref[:]
    c_col = cls_ref[:]
    x_row = out_ref[:].T
    c_row = c_col.T
    ids_row = ids_ref[:].reshape(1, B)
    ids_col = ids_row.T

    sin_t = jnp.sqrt(jnp.maximum(1.0 - x_row * x_row, 0.0))
    cos_m = jnp.float32(jnp.cos(ANG_MARGIN))
    sin_m = jnp.float32(jnp.sin(ANG_MARGIN))
    logits = (x_row * cos_m - (2.0 * c_row - 1.0) * sin_t * sin_m) * S_SCALE
    bce_terms = (jnp.maximum(logits, 0.0) - logits * c_row
                 + jnp.log1p(jnp.exp(-jnp.abs(logits))))
    bce = jnp.sum(bce_terms) * (1.0 / B)

    ones_d = jnp.ones((D, 1), dtype=jnp.float32)
    nrm = jax.lax.dot_general(emb * emb, ones_d, (((1,), (0,)), ((), ())),
                              preferred_element_type=jnp.float32)
    rsum = jax.lax.dot_general(emb, ones_d, (((1,), (0,)), ((), ())),
                               preferred_element_type=jnp.float32)
    aux1 = nrm + (2.0 * EPS) * rsum + (D * EPS * EPS)
    aux2 = nrm - (2.0 * EPS) * rsum
    ones_col = jnp.ones((B, 1), dtype=jnp.float32)
    u = jnp.concatenate([emb * -2.0, aux1, ones_col], axis=1)
    v = jnp.concatenate([emb, ones_col, aux2], axis=1)
    d2 = jax.lax.dot_general(u, v, (((1,), (1,)), ((), ())),
                             preferred_element_type=jnp.float32)
    dist = jnp.sqrt(jnp.maximum(d2, 0.0))

    id_eq = ids_col == ids_row
    cls_eq = c_col == c_row
    c1 = jnp.logical_and(~id_eq, cls_eq)
    c2 = jnp.logical_and(id_eq, ~cls_eq)

    s1 = jnp.sum(jnp.where(c1, dist, 0.0)) * 0.5
    n1 = jnp.sum(jnp.where(c1, 1.0, 0.0)) * 0.5
    s2 = jnp.sum(jnp.where(c2, jnp.maximum(0.0, M_MARGIN - dist), 0.0)) * 0.5
    n2 = jnp.sum(jnp.where(c2, 1.0, 0.0)) * 0.5

    t1 = s1 / jnp.maximum(n1, 1.0)
    t2 = s2 / jnp.maximum(n2, 1.0)
    loss = (GAMMA * bce
            + jnp.where(n1 > 0, ALPHA * t1, 0.0)
            + jnp.where(n2 > 0, BETA * t2, 0.0))
    res_ref[:] = jnp.full((1, 1), loss, dtype=jnp.float32)


def kernel(outputs, classes, emb, ids):
    res = pl.pallas_call(
        _loss_body,
        out_shape=jax.ShapeDtypeStruct((1, 1), jnp.float32),
    )(outputs, classes, emb, ids.astype(jnp.int32))
    return res.reshape(())


# speedup vs baseline: 776.2430x; 1.0094x over previous
import jax
import jax.numpy as jnp
from jax.experimental import pallas as pl
from jax.experimental.pallas import tpu as pltpu

ALPHA = 0.5
BETA = 0.5
GAMMA = 1.0
M_MARGIN = 2.0
S_SCALE = 64.0
ANG_MARGIN = 0.75
EPS = 1e-6
B = 512
D = 256


def _loss_body(out_ref, cls_ref, emb_ref, ids_ref, res_ref):
    emb = emb_ref[:]
    c_col = cls_ref[:]
    x_row = out_ref[:].T
    c_row = c_col.T
    ids_row = ids_ref[:].reshape(1, B)
    ids_col = ids_row.T

    sin_t = jnp.sqrt(jnp.maximum(1.0 - x_row * x_row, 0.0))
    cos_m = jnp.float32(jnp.cos(ANG_MARGIN))
    sin_m = jnp.float32(jnp.sin(ANG_MARGIN))
    logits = (x_row * cos_m - (2.0 * c_row - 1.0) * sin_t * sin_m) * S_SCALE
    bce_terms = (jnp.maximum(logits, 0.0) - logits * c_row
                 + jnp.log1p(jnp.exp(-jnp.abs(logits))))
    bce = jnp.sum(bce_terms) * (1.0 / B)

    ones_d = jnp.ones((D, 1), dtype=jnp.float32)
    nrm = jax.lax.dot_general(emb * emb, ones_d, (((1,), (0,)), ((), ())),
                              preferred_element_type=jnp.float32)
    rsum = jax.lax.dot_general(emb, ones_d, (((1,), (0,)), ((), ())),
                               preferred_element_type=jnp.float32)
    aux1 = nrm + (2.0 * EPS) * rsum + (D * EPS * EPS)
    aux2 = nrm - (2.0 * EPS) * rsum
    ones_col = jnp.ones((B, 1), dtype=jnp.float32)
    u = jnp.concatenate([emb * -2.0, aux1, ones_col], axis=1)
    v = jnp.concatenate([emb, ones_col, aux2], axis=1)
    d2 = jax.lax.dot_general(u, v, (((1,), (1,)), ((), ())),
                             preferred_element_type=jnp.float32)
    dist = jnp.sqrt(jnp.maximum(d2, 0.0))

    id_eq = ids_col == ids_row
    cls_eq = c_col == c_row

    s1 = jnp.sum(jnp.where(cls_eq, jnp.where(id_eq, 0.0, dist), 0.0)) * 0.5
    n1 = jnp.sum(jnp.where(cls_eq, jnp.where(id_eq, 0.0, 1.0), 0.0)) * 0.5
    hinge = jnp.maximum(0.0, M_MARGIN - dist)
    s2 = jnp.sum(jnp.where(cls_eq, 0.0, jnp.where(id_eq, hinge, 0.0))) * 0.5
    n2 = jnp.sum(jnp.where(cls_eq, 0.0, jnp.where(id_eq, 1.0, 0.0))) * 0.5

    t1 = s1 / jnp.maximum(n1, 1.0)
    t2 = s2 / jnp.maximum(n2, 1.0)
    loss = (GAMMA * bce
            + jnp.where(n1 > 0, ALPHA * t1, 0.0)
            + jnp.where(n2 > 0, BETA * t2, 0.0))
    res_ref[0] = loss


def kernel(outputs, classes, emb, ids):
    res = pl.pallas_call(
        _loss_body,
        out_shape=jax.ShapeDtypeStruct((1,), jnp.float32),
        out_specs=pl.BlockSpec(memory_space=pltpu.SMEM),
    )(outputs, classes, emb, ids.astype(jnp.int32))
    return res.reshape(())
